# baseline (speedup 1.0000x reference)
import functools

import jax
import jax.numpy as jnp
from jax import lax
from jax.experimental import pallas as pl
from jax.experimental.pallas import tpu as pltpu
from jax.experimental.pallas import tpu_sc as plsc

NC = 2
NS = 16
NW = NC * NS
C = 80
BCH = 25


def _dense1_body(x_ref, scls_ref, wfc0_ref, bfc0_ref, w1a_ref, w1b_ref,
                 as_row_ref, ad_row_ref, h_ref, as_ref, ad_ref):
    s = jnp.dot(scls_ref[...], wfc0_ref[...],
                preferred_element_type=jnp.float32) + bfc0_ref[...]
    sh = jnp.dot(s, w1b_ref[...], preferred_element_type=jnp.float32)
    h = jnp.dot(x_ref[...], w1a_ref[...],
                preferred_element_type=jnp.float32) + sh
    h_ref[...] = h
    as_ref[...] = jnp.sum(h * as_row_ref[...], axis=1, keepdims=True)
    ad_ref[...] = jnp.sum(h * ad_row_ref[...], axis=1, keepdims=True)


def _mid_body(accp_ref, denp_ref, h_ref, as_ref, ad_ref, b1_ref, w2_ref,
              as_row_ref, ad_row_ref, h2_ref, as2_ref, ad2_ref):
    acc = accp_ref[0] + accp_ref[1]
    den = jnp.sum(denp_ref[...], axis=0)[:, None]
    a = as_ref[...] + ad_ref[...]
    a = jnp.where(a >= 0.0, a, 0.2 * a)
    exself = jnp.exp(a)
    out = (acc + exself * h_ref[...]) / (den + exself) + b1_ref[...]
    hact = jnp.where(out > 0.0, out, jnp.exp(out) - 1.0)
    h2 = jnp.dot(hact, w2_ref[...], preferred_element_type=jnp.float32)
    h2_ref[...] = h2
    as2_ref[...] = jnp.sum(h2 * as_row_ref[...], axis=1, keepdims=True)
    ad2_ref[...] = jnp.sum(h2 * ad_row_ref[...], axis=1, keepdims=True)


def _fin_body(accp_ref, denp_ref, h_ref, as_ref, ad_ref, b2_ref,
              wfa_ref, wfb_ref, bf2_ref, a_out_ref, b_out_ref):
    acc = accp_ref[0] + accp_ref[1]
    den = jnp.sum(denp_ref[...], axis=0)[:, None]
    a = as_ref[...] + ad_ref[...]
    a = jnp.where(a >= 0.0, a, 0.2 * a)
    exself = jnp.exp(a)
    out = (acc + exself * h_ref[...]) / (den + exself) + b2_ref[...]
    a_out_ref[...] = jnp.dot(out, wfa_ref[...],
                             preferred_element_type=jnp.float32) + bf2_ref[...]
    b_out_ref[...] = jnp.dot(out, wfb_ref[...],
                             preferred_element_type=jnp.float32)


def _gat_scalar_body(asrc_hbm, adst_hbm, src_hbm, dst_hbm,
                     ex_out, den_out,
                     asrc_v, adst_v, den_v, idx_s, idx_d, ex_v):
    cid = lax.axis_index("c")
    sid = lax.axis_index("s")
    wid = cid * NS + sid
    n = asrc_v.shape[0]
    ept = idx_s.shape[0]

    pltpu.sync_copy(asrc_hbm, asrc_v)
    pltpu.sync_copy(adst_hbm, adst_v)
    pltpu.sync_copy(src_hbm.at[pl.ds(wid * ept, ept)], idx_s)
    pltpu.sync_copy(dst_hbm.at[pl.ds(wid * ept, ept)], idx_d)

    zeros16 = jnp.zeros((16,), jnp.float32)

    def zden(i, carry):
        den_v[pl.ds(i * 16, 16)] = zeros16
        return carry

    lax.fori_loop(0, n // 16, zden, 0)

    def grp(i, carry):
        si = idx_s[pl.ds(i * 16, 16)]
        di = idx_d[pl.ds(i * 16, 16)]
        av = plsc.load_gather(asrc_v, [si])
        dv = plsc.load_gather(adst_v, [di])
        al = av + dv
        al = jnp.where(al >= 0.0, al, al * 0.2)
        ex = jnp.exp(al)
        ex_v[pl.ds(i * 16, 16)] = ex
        plsc.addupdate_scatter(den_v, [di], ex)
        return carry

    lax.fori_loop(0, ept // 16, grp, 0)

    pltpu.sync_copy(ex_v, ex_out.at[pl.ds(wid * ept, ept)])
    pltpu.sync_copy(den_v, den_out.at[pl.ds(wid * n, n)])


def _gat_agg_body(h_hbm, ex_hbm, src_hbm, dst_hbm,
                  acc_out,
                  idx_s, idx_d, ex_b, rows_v, acc_sh):
    cid = lax.axis_index("c")
    sid = lax.axis_index("s")
    wid = cid * NS + sid
    n, d = acc_sh.shape
    nblk = src_hbm.shape[1]
    b8 = (n // NS) // 8 * 8
    tail = n - NS * b8

    zeros16 = jnp.zeros((16,), jnp.float32)

    def zrow(c, carry):
        for j in range(d // 16):
            rows_v[c, pl.ds(j * 16, 16)] = zeros16
        return carry

    lax.fori_loop(0, C, zrow, 0)

    full = b8 // C
    rem = b8 - full * C
    for z in range(full):
        pltpu.sync_copy(rows_v, acc_sh.at[pl.ds(sid * b8 + z * C, C)])
    if rem:
        pltpu.sync_copy(rows_v.at[pl.ds(0, rem)],
                        acc_sh.at[pl.ds(sid * b8 + full * C, rem)])
    if tail:
        @pl.when(sid == 0)
        def _zero_tail():
            pltpu.sync_copy(rows_v.at[pl.ds(0, tail)],
                            acc_sh.at[pl.ds(NS * b8, tail)])
    plsc.subcore_barrier()

    def blk(b, carry):
        pltpu.sync_copy(src_hbm.at[wid, b], idx_s)
        pltpu.sync_copy(dst_hbm.at[wid, b], idx_d)
        pltpu.sync_copy(ex_hbm.at[wid, b], ex_b)

        def chunk(k, kcarry):
            pltpu.sync_copy(h_hbm.at[idx_s.at[k]], rows_v)

            def sgrp(i, scarry):
                ex16 = ex_b[k, pl.ds(i * 16, 16)]
                for jj in range(16):
                    e = ex16[jj]
                    c = i * 16 + jj
                    for j in range(d // 16):
                        rows_v[c, pl.ds(j * 16, 16)] = (
                            rows_v[c, pl.ds(j * 16, 16)] * e)
                return scarry

            lax.fori_loop(0, C // 16, sgrp, 0)

            pltpu.sync_copy(rows_v, acc_sh.at[idx_d.at[k]], add=True)
            return kcarry

        lax.fori_loop(0, BCH, chunk, 0)
        return carry

    lax.fori_loop(0, nblk, blk, 0)

    plsc.subcore_barrier()
    pltpu.sync_copy(acc_sh.at[pl.ds(sid * b8, b8)],
                    acc_out.at[cid, pl.ds(sid * b8, b8)])
    if tail:
        @pl.when(sid == 0)
        def _write_tail():
            pltpu.sync_copy(acc_sh.at[pl.ds(NS * b8, tail)],
                            acc_out.at[cid, pl.ds(NS * b8, tail)])


def _edge_mlp_body(a_hbm, b_hbm, src_hbm, dst_hbm, w3_hbm, b3_hbm,
                   out_hbm,
                   idx_s, idx_d, rows_a, rows_b, w3_v, b3_v, logit_v):
    cid = lax.axis_index("c")
    sid = lax.axis_index("s")
    wid = cid * NS + sid
    d = rows_a.shape[1]
    k_chunks = idx_s.shape[0]
    ept = k_chunks * C

    pltpu.sync_copy(src_hbm.at[wid], idx_s)
    pltpu.sync_copy(dst_hbm.at[wid], idx_d)
    pltpu.sync_copy(w3_hbm, w3_v)
    pltpu.sync_copy(b3_hbm, b3_v)

    w3s = []
    for j in range(d // 16):
        w16 = w3_v[pl.ds(j * 16, 16)]
        for l in range(16):
            w3s.append(w16[l])
    b3vec = b3_v[...]

    def chunk(k, carry):
        pltpu.sync_copy(a_hbm.at[idx_s.at[k]], rows_a)
        pltpu.sync_copy(b_hbm.at[idx_d.at[k]], rows_b)

        def grp(i, gcarry):
            rlane = lax.iota(jnp.int32, 16) + i * 16
            accs = [jnp.zeros((16,), jnp.float32) for _ in range(8)]
            for col in range(d):
                cvec = jnp.full((16,), col, jnp.int32)
                va = plsc.load_gather(rows_a, [rlane, cvec])
                vb = plsc.load_gather(rows_b, [rlane, cvec])
                accs[col % 8] = (accs[col % 8]
                                 + jnp.maximum(va + vb, 0.0) * w3s[col])
            acc = ((accs[0] + accs[1]) + (accs[2] + accs[3])
                   + ((accs[4] + accs[5]) + (accs[6] + accs[7])))
            lg = acc + b3vec
            logit_v[pl.ds(i * 16, 16)] = 1.0 / (1.0 + jnp.exp(-lg))
            return gcarry

        lax.fori_loop(0, C // 16, grp, 0)

        pltpu.sync_copy(logit_v, out_hbm.at[pl.ds(wid * ept + k * C, C)])
        return carry

    lax.fori_loop(0, k_chunks, chunk, 0)


def kernel(x, edge_index, sentence_cls, W_fc0, b_fc0, W1, att_src1, att_dst1,
           b1, W2, att_src2, att_dst2, b2, W_fc2, b_fc2, W_fc3, b_fc3):
    n, d = x.shape
    e = edge_index.shape[1]
    ept = e // NW
    kc = ept // C
    nblk = kc // BCH
    assert e == NW * ept and ept == kc * C and kc == nblk * BCH
    assert n % NS == 0 and n % 16 == 0 and d % 16 == 0 and ept % 16 == 0

    src = edge_index[0].astype(jnp.int32)
    dst = edge_index[1].astype(jnp.int32)
    src4 = src.reshape(NW, nblk, BCH, C)
    dst4 = dst.reshape(NW, nblk, BCH, C)
    src_w = src.reshape(NW, kc, C)
    dst_w = dst.reshape(NW, kc, C)
    f32 = jnp.float32

    mesh = plsc.VectorSubcoreMesh(core_axis_name="c", subcore_axis_name="s")
    sc_params = pltpu.CompilerParams(needs_layout_passes=False)

    gat_scalar = functools.partial(
        pl.kernel,
        out_type=[jax.ShapeDtypeStruct((e,), f32),
                  jax.ShapeDtypeStruct((NW * n,), f32)],
        mesh=mesh,
        compiler_params=sc_params,
        scratch_types=[pltpu.VMEM((n,), f32),
                       pltpu.VMEM((n,), f32),
                       pltpu.VMEM((n,), f32),
                       pltpu.VMEM((ept,), jnp.int32),
                       pltpu.VMEM((ept,), jnp.int32),
                       pltpu.VMEM((ept,), f32)],
    )(_gat_scalar_body)

    gat_agg = functools.partial(
        pl.kernel,
        out_type=jax.ShapeDtypeStruct((NC, n, d), f32),
        mesh=mesh,
        compiler_params=sc_params,
        scratch_types=[pltpu.VMEM((BCH, C), jnp.int32),
                       pltpu.VMEM((BCH, C), jnp.int32),
                       pltpu.VMEM((BCH, C), f32),
                       pltpu.VMEM((C, d), f32),
                       pltpu.VMEM_SHARED((n, d), f32)],
    )(_gat_agg_body)

    edge_mlp = functools.partial(
        pl.kernel,
        out_type=jax.ShapeDtypeStruct((e,), f32),
        mesh=mesh,
        compiler_params=sc_params,
        scratch_types=[pltpu.VMEM((kc, C), jnp.int32),
                       pltpu.VMEM((kc, C), jnp.int32),
                       pltpu.VMEM((C, d), f32),
                       pltpu.VMEM((C, d), f32),
                       pltpu.VMEM((d,), f32),
                       pltpu.VMEM((16,), f32),
                       pltpu.VMEM((C,), f32)],
    )(_edge_mlp_body)

    def gat_layer(h, a_s, a_d):
        ex, den = gat_scalar(a_s.reshape(-1), a_d.reshape(-1), src, dst)
        ex4 = ex.reshape(NW, nblk, BCH, C)
        acc = gat_agg(h, ex4, src4, dst4)
        return acc, den.reshape(NW, n)

    h1, as1, ad1 = pl.pallas_call(
        _dense1_body,
        out_shape=[jax.ShapeDtypeStruct((n, d), f32),
                   jax.ShapeDtypeStruct((n, 1), f32),
                   jax.ShapeDtypeStruct((n, 1), f32)],
    )(x, sentence_cls.reshape(1, -1), W_fc0, b_fc0.reshape(1, -1),
      W1[:d], W1[d:], att_src1.reshape(1, -1), att_dst1.reshape(1, -1))

    acc1, den1 = gat_layer(h1, as1, ad1)

    h2, as2, ad2 = pl.pallas_call(
        _mid_body,
        out_shape=[jax.ShapeDtypeStruct((n, d), f32),
                   jax.ShapeDtypeStruct((n, 1), f32),
                   jax.ShapeDtypeStruct((n, 1), f32)],
    )(acc1, den1, h1, as1, ad1, b1.reshape(1, -1), W2,
      att_src2.reshape(1, -1), att_dst2.reshape(1, -1))

    acc2, den2 = gat_layer(h2, as2, ad2)

    a_tab, b_tab = pl.pallas_call(
        _fin_body,
        out_shape=[jax.ShapeDtypeStruct((n, d), f32),
                   jax.ShapeDtypeStruct((n, d), f32)],
    )(acc2, den2, h2, as2, ad2, b2.reshape(1, -1),
      W_fc2[:d], W_fc2[d:], b_fc2.reshape(1, -1))

    probs = edge_mlp(a_tab, b_tab, src_w, dst_w, W_fc3.reshape(-1),
                     jnp.broadcast_to(b_fc3, (16,)))
    return probs.reshape(e, 1)


# speedup vs baseline: 2.3774x; 2.3774x over previous
import functools

import jax
import jax.numpy as jnp
from jax import lax
from jax.experimental import pallas as pl
from jax.experimental.pallas import tpu as pltpu
from jax.experimental.pallas import tpu_sc as plsc

NC = 2
NS = 16
NW = NC * NS
C = 80
BCH = 25


def _dense1_body(x_ref, scls_ref, wfc0_ref, bfc0_ref, w1a_ref, w1b_ref,
                 as_row_ref, ad_row_ref, h_ref, as_ref, ad_ref):
    s = jnp.dot(scls_ref[...], wfc0_ref[...],
                preferred_element_type=jnp.float32) + bfc0_ref[...]
    sh = jnp.dot(s, w1b_ref[...], preferred_element_type=jnp.float32)
    h = jnp.dot(x_ref[...], w1a_ref[...],
                preferred_element_type=jnp.float32) + sh
    h_ref[...] = h
    as_ref[...] = jnp.sum(h * as_row_ref[...], axis=1, keepdims=True)
    ad_ref[...] = jnp.sum(h * ad_row_ref[...], axis=1, keepdims=True)


def _mid_body(accp_ref, denp_ref, h_ref, as_ref, ad_ref, b1_ref, w2_ref,
              as_row_ref, ad_row_ref, h2_ref, as2_ref, ad2_ref):
    acc = accp_ref[0] + accp_ref[1]
    den = jnp.sum(denp_ref[...], axis=0)[:, None]
    a = as_ref[...] + ad_ref[...]
    a = jnp.where(a >= 0.0, a, 0.2 * a)
    exself = jnp.exp(a)
    out = (acc + exself * h_ref[...]) / (den + exself) + b1_ref[...]
    hact = jnp.where(out > 0.0, out, jnp.exp(out) - 1.0)
    h2 = jnp.dot(hact, w2_ref[...], preferred_element_type=jnp.float32)
    h2_ref[...] = h2
    as2_ref[...] = jnp.sum(h2 * as_row_ref[...], axis=1, keepdims=True)
    ad2_ref[...] = jnp.sum(h2 * ad_row_ref[...], axis=1, keepdims=True)


def _fin_body(accp_ref, denp_ref, h_ref, as_ref, ad_ref, b2_ref,
              wfa_ref, wfb_ref, bf2_ref, a_out_ref, b_out_ref):
    acc = accp_ref[0] + accp_ref[1]
    den = jnp.sum(denp_ref[...], axis=0)[:, None]
    a = as_ref[...] + ad_ref[...]
    a = jnp.where(a >= 0.0, a, 0.2 * a)
    exself = jnp.exp(a)
    out = (acc + exself * h_ref[...]) / (den + exself) + b2_ref[...]
    a_out_ref[...] = jnp.dot(out, wfa_ref[...],
                             preferred_element_type=jnp.float32) + bf2_ref[...]
    b_out_ref[...] = jnp.dot(out, wfb_ref[...],
                             preferred_element_type=jnp.float32)


def _gat_scalar_body(asrc_hbm, adst_hbm, src_hbm, dst_hbm,
                     ex_out, den_out,
                     asrc_v, adst_v, den_v, idx_s, idx_d, ex_v):
    cid = lax.axis_index("c")
    sid = lax.axis_index("s")
    wid = cid * NS + sid
    n = asrc_v.shape[0]
    ept = idx_s.shape[0]

    pltpu.sync_copy(asrc_hbm, asrc_v)
    pltpu.sync_copy(adst_hbm, adst_v)
    pltpu.sync_copy(src_hbm.at[pl.ds(wid * ept, ept)], idx_s)
    pltpu.sync_copy(dst_hbm.at[pl.ds(wid * ept, ept)], idx_d)

    zeros16 = jnp.zeros((16,), jnp.float32)

    def zden(i, carry):
        den_v[pl.ds(i * 16, 16)] = zeros16
        return carry

    lax.fori_loop(0, n // 16, zden, 0)

    def grp(i, carry):
        si = idx_s[pl.ds(i * 16, 16)]
        di = idx_d[pl.ds(i * 16, 16)]
        av = plsc.load_gather(asrc_v, [si])
        dv = plsc.load_gather(adst_v, [di])
        al = av + dv
        al = jnp.where(al >= 0.0, al, al * 0.2)
        ex = jnp.exp(al)
        ex_v[pl.ds(i * 16, 16)] = ex
        plsc.addupdate_scatter(den_v, [di], ex)
        return carry

    lax.fori_loop(0, ept // 16, grp, 0)

    pltpu.sync_copy(ex_v, ex_out.at[pl.ds(wid * ept, ept)])
    pltpu.sync_copy(den_v, den_out.at[pl.ds(wid * n, n)])


def _gat_agg_body(h_hbm, ex_hbm, src_hbm, dst_hbm,
                  acc_out,
                  idx_s, idx_d, ex_b, rows_v, acc_sh):
    cid = lax.axis_index("c")
    sid = lax.axis_index("s")
    wid = cid * NS + sid
    n, d = acc_sh.shape
    nblk = src_hbm.shape[1]
    b8 = (n // NS) // 8 * 8
    tail = n - NS * b8

    zeros16 = jnp.zeros((16,), jnp.float32)

    def zrow(c, carry):
        for j in range(d // 16):
            rows_v[c, pl.ds(j * 16, 16)] = zeros16
        return carry

    lax.fori_loop(0, C, zrow, 0)

    full = b8 // C
    rem = b8 - full * C
    for z in range(full):
        pltpu.sync_copy(rows_v, acc_sh.at[pl.ds(sid * b8 + z * C, C)])
    if rem:
        pltpu.sync_copy(rows_v.at[pl.ds(0, rem)],
                        acc_sh.at[pl.ds(sid * b8 + full * C, rem)])
    if tail:
        @pl.when(sid == 0)
        def _zero_tail():
            pltpu.sync_copy(rows_v.at[pl.ds(0, tail)],
                            acc_sh.at[pl.ds(NS * b8, tail)])
    plsc.subcore_barrier()

    def blk(b, carry):
        pltpu.sync_copy(src_hbm.at[wid, b], idx_s)
        pltpu.sync_copy(dst_hbm.at[wid, b], idx_d)
        pltpu.sync_copy(ex_hbm.at[wid, b], ex_b)

        def chunk(k, kcarry):
            pltpu.sync_copy(h_hbm.at[idx_s.at[k]], rows_v)

            def sgrp(i, scarry):
                ex16 = ex_b[k, pl.ds(i * 16, 16)]
                for jj in range(16):
                    e = ex16[jj]
                    c = i * 16 + jj
                    for j in range(d // 16):
                        rows_v[c, pl.ds(j * 16, 16)] = (
                            rows_v[c, pl.ds(j * 16, 16)] * e)
                return scarry

            lax.fori_loop(0, C // 16, sgrp, 0)

            pltpu.sync_copy(rows_v, acc_sh.at[idx_d.at[k]], add=True)
            return kcarry

        lax.fori_loop(0, BCH, chunk, 0)
        return carry

    lax.fori_loop(0, nblk, blk, 0)

    plsc.subcore_barrier()
    pltpu.sync_copy(acc_sh.at[pl.ds(sid * b8, b8)],
                    acc_out.at[cid, pl.ds(sid * b8, b8)])
    if tail:
        @pl.when(sid == 0)
        def _write_tail():
            pltpu.sync_copy(acc_sh.at[pl.ds(NS * b8, tail)],
                            acc_out.at[cid, pl.ds(NS * b8, tail)])


def _edge_mlp_body(a_hbm, b_hbm, src_hbm, dst_hbm, w3_hbm, b3_hbm,
                   out_hbm,
                   idx_s, idx_d, rows_a, rows_b, w3_v, b3_v, logit_v):
    cid = lax.axis_index("c")
    sid = lax.axis_index("s")
    wid = cid * NS + sid
    d = rows_a.shape[1]
    k_chunks = idx_s.shape[0]
    ept = k_chunks * C

    pltpu.sync_copy(src_hbm.at[wid], idx_s)
    pltpu.sync_copy(dst_hbm.at[wid], idx_d)
    pltpu.sync_copy(w3_hbm, w3_v)
    pltpu.sync_copy(b3_hbm, b3_v)

    w3v = [w3_v[pl.ds(j * 16, 16)] for j in range(d // 16)]
    b3vec = b3_v[...]
    iota16 = lax.iota(jnp.int32, 16)
    perms = [iota16 ^ 8, iota16 ^ 4, iota16 ^ 2, iota16 ^ 1]
    masks = [iota16 == jj for jj in range(16)]

    def chunk(k, carry):
        pltpu.sync_copy(a_hbm.at[idx_s.at[k]], rows_a)
        pltpu.sync_copy(b_hbm.at[idx_d.at[k]], rows_b)

        def grp(i, gcarry):
            lg = b3vec
            for jj in range(16):
                c = i * 16 + jj
                acc0 = jnp.zeros((16,), jnp.float32)
                acc1 = jnp.zeros((16,), jnp.float32)
                for j in range(d // 16):
                    va = rows_a[c, pl.ds(j * 16, 16)]
                    vb = rows_b[c, pl.ds(j * 16, 16)]
                    v = jnp.maximum(va + vb, 0.0) * w3v[j]
                    if j % 2 == 0:
                        acc0 = acc0 + v
                    else:
                        acc1 = acc1 + v
                r = acc0 + acc1
                for p in perms:
                    r = r + lax.gather(
                        r, p[:, None],
                        lax.GatherDimensionNumbers(
                            offset_dims=(), collapsed_slice_dims=(0,),
                            start_index_map=(0,)),
                        slice_sizes=(1,),
                        mode=lax.GatherScatterMode.PROMISE_IN_BOUNDS)
                lg = jnp.where(masks[jj], r + b3vec, lg)
            logit_v[pl.ds(i * 16, 16)] = 1.0 / (1.0 + jnp.exp(-lg))
            return gcarry

        lax.fori_loop(0, C // 16, grp, 0)

        pltpu.sync_copy(logit_v, out_hbm.at[pl.ds(wid * ept + k * C, C)])
        return carry

    lax.fori_loop(0, k_chunks, chunk, 0)


def kernel(x, edge_index, sentence_cls, W_fc0, b_fc0, W1, att_src1, att_dst1,
           b1, W2, att_src2, att_dst2, b2, W_fc2, b_fc2, W_fc3, b_fc3):
    n, d = x.shape
    e = edge_index.shape[1]
    ept = e // NW
    kc = ept // C
    nblk = kc // BCH
    assert e == NW * ept and ept == kc * C and kc == nblk * BCH
    assert n % NS == 0 and n % 16 == 0 and d % 16 == 0 and ept % 16 == 0

    src = edge_index[0].astype(jnp.int32)
    dst = edge_index[1].astype(jnp.int32)
    src4 = src.reshape(NW, nblk, BCH, C)
    dst4 = dst.reshape(NW, nblk, BCH, C)
    src_w = src.reshape(NW, kc, C)
    dst_w = dst.reshape(NW, kc, C)
    f32 = jnp.float32

    mesh = plsc.VectorSubcoreMesh(core_axis_name="c", subcore_axis_name="s")
    sc_params = pltpu.CompilerParams(needs_layout_passes=False)

    gat_scalar = functools.partial(
        pl.kernel,
        out_type=[jax.ShapeDtypeStruct((e,), f32),
                  jax.ShapeDtypeStruct((NW * n,), f32)],
        mesh=mesh,
        compiler_params=sc_params,
        scratch_types=[pltpu.VMEM((n,), f32),
                       pltpu.VMEM((n,), f32),
                       pltpu.VMEM((n,), f32),
                       pltpu.VMEM((ept,), jnp.int32),
                       pltpu.VMEM((ept,), jnp.int32),
                       pltpu.VMEM((ept,), f32)],
    )(_gat_scalar_body)

    gat_agg = functools.partial(
        pl.kernel,
        out_type=jax.ShapeDtypeStruct((NC, n, d), f32),
        mesh=mesh,
        compiler_params=sc_params,
        scratch_types=[pltpu.VMEM((BCH, C), jnp.int32),
                       pltpu.VMEM((BCH, C), jnp.int32),
                       pltpu.VMEM((BCH, C), f32),
                       pltpu.VMEM((C, d), f32),
                       pltpu.VMEM_SHARED((n, d), f32)],
    )(_gat_agg_body)

    edge_mlp = functools.partial(
        pl.kernel,
        out_type=jax.ShapeDtypeStruct((e,), f32),
        mesh=mesh,
        compiler_params=sc_params,
        scratch_types=[pltpu.VMEM((kc, C), jnp.int32),
                       pltpu.VMEM((kc, C), jnp.int32),
                       pltpu.VMEM((C, d), f32),
                       pltpu.VMEM((C, d), f32),
                       pltpu.VMEM((d,), f32),
                       pltpu.VMEM((16,), f32),
                       pltpu.VMEM((C,), f32)],
    )(_edge_mlp_body)

    def gat_layer(h, a_s, a_d):
        ex, den = gat_scalar(a_s.reshape(-1), a_d.reshape(-1), src, dst)
        ex4 = ex.reshape(NW, nblk, BCH, C)
        acc = gat_agg(h, ex4, src4, dst4)
        return acc, den.reshape(NW, n)

    h1, as1, ad1 = pl.pallas_call(
        _dense1_body,
        out_shape=[jax.ShapeDtypeStruct((n, d), f32),
                   jax.ShapeDtypeStruct((n, 1), f32),
                   jax.ShapeDtypeStruct((n, 1), f32)],
    )(x, sentence_cls.reshape(1, -1), W_fc0, b_fc0.reshape(1, -1),
      W1[:d], W1[d:], att_src1.reshape(1, -1), att_dst1.reshape(1, -1))

    acc1, den1 = gat_layer(h1, as1, ad1)

    h2, as2, ad2 = pl.pallas_call(
        _mid_body,
        out_shape=[jax.ShapeDtypeStruct((n, d), f32),
                   jax.ShapeDtypeStruct((n, 1), f32),
                   jax.ShapeDtypeStruct((n, 1), f32)],
    )(acc1, den1, h1, as1, ad1, b1.reshape(1, -1), W2,
      att_src2.reshape(1, -1), att_dst2.reshape(1, -1))

    acc2, den2 = gat_layer(h2, as2, ad2)

    a_tab, b_tab = pl.pallas_call(
        _fin_body,
        out_shape=[jax.ShapeDtypeStruct((n, d), f32),
                   jax.ShapeDtypeStruct((n, d), f32)],
    )(acc2, den2, h2, as2, ad2, b2.reshape(1, -1),
      W_fc2[:d], W_fc2[d:], b_fc2.reshape(1, -1))

    probs = edge_mlp(a_tab, b_tab, src_w, dst_w, W_fc3.reshape(-1),
                     jnp.broadcast_to(b_fc3, (16,)))
    return probs.reshape(e, 1)


# speedup vs baseline: 3.5961x; 1.5126x over previous
import functools

import jax
import jax.numpy as jnp
from jax import lax
from jax.experimental import pallas as pl
from jax.experimental.pallas import tpu as pltpu
from jax.experimental.pallas import tpu_sc as plsc

NC = 2
NS = 16
NW = NC * NS
C = 80
BCH = 25


def _dense1_body(x_ref, scls_ref, wfc0_ref, bfc0_ref, w1a_ref, w1b_ref,
                 as_row_ref, ad_row_ref, h_ref, as_ref, ad_ref):
    s = jnp.dot(scls_ref[...], wfc0_ref[...],
                preferred_element_type=jnp.float32) + bfc0_ref[...]
    sh = jnp.dot(s, w1b_ref[...], preferred_element_type=jnp.float32)
    h = jnp.dot(x_ref[...], w1a_ref[...],
                preferred_element_type=jnp.float32) + sh
    h_ref[...] = h
    as_ref[...] = jnp.sum(h * as_row_ref[...], axis=1, keepdims=True)
    ad_ref[...] = jnp.sum(h * ad_row_ref[...], axis=1, keepdims=True)


def _mid_body(accp_ref, denp_ref, h_ref, as_ref, ad_ref, b1_ref, w2_ref,
              as_row_ref, ad_row_ref, h2_ref, as2_ref, ad2_ref):
    acc = accp_ref[0] + accp_ref[1]
    den = jnp.sum(denp_ref[...], axis=0)[:, None]
    a = as_ref[...] + ad_ref[...]
    a = jnp.where(a >= 0.0, a, 0.2 * a)
    exself = jnp.exp(a)
    out = (acc + exself * h_ref[...]) / (den + exself) + b1_ref[...]
    hact = jnp.where(out > 0.0, out, jnp.exp(out) - 1.0)
    h2 = jnp.dot(hact, w2_ref[...], preferred_element_type=jnp.float32)
    h2_ref[...] = h2
    as2_ref[...] = jnp.sum(h2 * as_row_ref[...], axis=1, keepdims=True)
    ad2_ref[...] = jnp.sum(h2 * ad_row_ref[...], axis=1, keepdims=True)


def _fin_body(accp_ref, denp_ref, h_ref, as_ref, ad_ref, b2_ref,
              wfa_ref, wfb_ref, bf2_ref, a_out_ref, b_out_ref):
    acc = accp_ref[0] + accp_ref[1]
    den = jnp.sum(denp_ref[...], axis=0)[:, None]
    a = as_ref[...] + ad_ref[...]
    a = jnp.where(a >= 0.0, a, 0.2 * a)
    exself = jnp.exp(a)
    out = (acc + exself * h_ref[...]) / (den + exself) + b2_ref[...]
    a_out_ref[...] = jnp.dot(out, wfa_ref[...],
                             preferred_element_type=jnp.float32) + bf2_ref[...]
    b_out_ref[...] = jnp.dot(out, wfb_ref[...],
                             preferred_element_type=jnp.float32)


def _gat_scalar_body(asrc_hbm, adst_hbm, src_hbm, dst_hbm,
                     ex_out, den_out,
                     asrc_v, adst_v, den_v, idx_s, idx_d, ex_v):
    cid = lax.axis_index("c")
    sid = lax.axis_index("s")
    wid = cid * NS + sid
    n = asrc_v.shape[0]
    ept = idx_s.shape[0]

    pltpu.sync_copy(asrc_hbm, asrc_v)
    pltpu.sync_copy(adst_hbm, adst_v)
    pltpu.sync_copy(src_hbm.at[pl.ds(wid * ept, ept)], idx_s)
    pltpu.sync_copy(dst_hbm.at[pl.ds(wid * ept, ept)], idx_d)

    zeros16 = jnp.zeros((16,), jnp.float32)

    def zden(i, carry):
        den_v[pl.ds(i * 16, 16)] = zeros16
        return carry

    lax.fori_loop(0, n // 16, zden, 0)

    def grp(i, carry):
        si = idx_s[pl.ds(i * 16, 16)]
        di = idx_d[pl.ds(i * 16, 16)]
        av = plsc.load_gather(asrc_v, [si])
        dv = plsc.load_gather(adst_v, [di])
        al = av + dv
        al = jnp.where(al >= 0.0, al, al * 0.2)
        ex = jnp.exp(al)
        ex_v[pl.ds(i * 16, 16)] = ex
        plsc.addupdate_scatter(den_v, [di], ex)
        return carry

    lax.fori_loop(0, ept // 16, grp, 0)

    pltpu.sync_copy(ex_v, ex_out.at[pl.ds(wid * ept, ept)])
    pltpu.sync_copy(den_v, den_out.at[pl.ds(wid * n, n)])


def _gat_agg_body(h_hbm, ex_hbm, src_hbm, dst_hbm,
                  acc_out,
                  idx_s, idx_d, ex_b, rows0, rows1, sem0, sem1, acc_sh):
    cid = lax.axis_index("c")
    sid = lax.axis_index("s")
    wid = cid * NS + sid
    n, d = acc_sh.shape
    nblk = src_hbm.shape[1]
    b8 = (n // NS) // 8 * 8
    tail = n - NS * b8

    zeros16 = jnp.zeros((16,), jnp.float32)

    def zrow(c, carry):
        for j in range(d // 16):
            rows0[c, pl.ds(j * 16, 16)] = zeros16
        return carry

    lax.fori_loop(0, C, zrow, 0)

    full = b8 // C
    rem = b8 - full * C
    for z in range(full):
        pltpu.sync_copy(rows0, acc_sh.at[pl.ds(sid * b8 + z * C, C)])
    if rem:
        pltpu.sync_copy(rows0.at[pl.ds(0, rem)],
                        acc_sh.at[pl.ds(sid * b8 + full * C, rem)])
    if tail:
        @pl.when(sid == 0)
        def _zero_tail():
            pltpu.sync_copy(rows0.at[pl.ds(0, tail)],
                            acc_sh.at[pl.ds(NS * b8, tail)])
    plsc.subcore_barrier()

    def start_g(k, rows, sem):
        pltpu.async_copy(h_hbm.at[idx_s.at[k]], rows, sem)

    def wait_g(rows, sem):
        pltpu.make_async_copy(h_hbm.at[idx_s.at[0]], rows, sem).wait()

    def do_chunk(k, rows):
        def sgrp(i, scarry):
            ex16 = ex_b[k, pl.ds(i * 16, 16)]
            for jj in range(16):
                e = ex16[jj]
                c = i * 16 + jj
                for j in range(d // 16):
                    rows[c, pl.ds(j * 16, 16)] = (
                        rows[c, pl.ds(j * 16, 16)] * e)
            return scarry

        lax.fori_loop(0, C // 16, sgrp, 0)
        pltpu.sync_copy(rows, acc_sh.at[idx_d.at[k]], add=True)

    def blk(b, carry):
        pltpu.sync_copy(src_hbm.at[wid, b], idx_s)
        pltpu.sync_copy(dst_hbm.at[wid, b], idx_d)
        pltpu.sync_copy(ex_hbm.at[wid, b], ex_b)

        start_g(0, rows0, sem0)

        def pair(p, pcarry):
            ka = 2 * p
            start_g(ka + 1, rows1, sem1)
            wait_g(rows0, sem0)
            do_chunk(ka, rows0)
            start_g(ka + 2, rows0, sem0)
            wait_g(rows1, sem1)
            do_chunk(ka + 1, rows1)
            return pcarry

        lax.fori_loop(0, (BCH - 3) // 2, pair, 0)

        ka = BCH - 3
        start_g(ka + 1, rows1, sem1)
        wait_g(rows0, sem0)
        do_chunk(ka, rows0)
        start_g(ka + 2, rows0, sem0)
        wait_g(rows1, sem1)
        do_chunk(ka + 1, rows1)
        wait_g(rows0, sem0)
        do_chunk(ka + 2, rows0)
        return carry

    lax.fori_loop(0, nblk, blk, 0)

    plsc.subcore_barrier()
    pltpu.sync_copy(acc_sh.at[pl.ds(sid * b8, b8)],
                    acc_out.at[cid, pl.ds(sid * b8, b8)])
    if tail:
        @pl.when(sid == 0)
        def _write_tail():
            pltpu.sync_copy(acc_sh.at[pl.ds(NS * b8, tail)],
                            acc_out.at[cid, pl.ds(NS * b8, tail)])


def _edge_mlp_body(a_hbm, b_hbm, src_hbm, dst_hbm, w3_hbm, b3_hbm,
                   out_hbm,
                   idx_s, idx_d, ra0, rb0, ra1, rb1, sem0, sem1,
                   w3_v, b3_v, logit_v):
    cid = lax.axis_index("c")
    sid = lax.axis_index("s")
    wid = cid * NS + sid
    d = ra0.shape[1]
    k_chunks = idx_s.shape[0]
    ept = k_chunks * C

    pltpu.sync_copy(src_hbm.at[wid], idx_s)
    pltpu.sync_copy(dst_hbm.at[wid], idx_d)
    pltpu.sync_copy(w3_hbm, w3_v)
    pltpu.sync_copy(b3_hbm, b3_v)

    w3v = [w3_v[pl.ds(j * 16, 16)] for j in range(d // 16)]
    b3vec = b3_v[...]
    iota16 = lax.iota(jnp.int32, 16)
    perms = [iota16 ^ 8, iota16 ^ 4, iota16 ^ 2, iota16 ^ 1]
    masks = [iota16 == jj for jj in range(16)]

    def start_g(k, ra, rb, sem):
        pltpu.async_copy(a_hbm.at[idx_s.at[k]], ra, sem)
        pltpu.async_copy(b_hbm.at[idx_d.at[k]], rb, sem)

    def wait_g(ra, rb, sem):
        pltpu.make_async_copy(a_hbm.at[idx_s.at[0]], ra, sem).wait()
        pltpu.make_async_copy(b_hbm.at[idx_d.at[0]], rb, sem).wait()

    def do_chunk(k, ra, rb):
        def grp(i, gcarry):
            lg = b3vec
            for jj in range(16):
                c = i * 16 + jj
                acc0 = jnp.zeros((16,), jnp.float32)
                acc1 = jnp.zeros((16,), jnp.float32)
                for j in range(d // 16):
                    va = ra[c, pl.ds(j * 16, 16)]
                    vb = rb[c, pl.ds(j * 16, 16)]
                    v = jnp.maximum(va + vb, 0.0) * w3v[j]
                    if j % 2 == 0:
                        acc0 = acc0 + v
                    else:
                        acc1 = acc1 + v
                r = acc0 + acc1
                for p in perms:
                    r = r + lax.gather(
                        r, p[:, None],
                        lax.GatherDimensionNumbers(
                            offset_dims=(), collapsed_slice_dims=(0,),
                            start_index_map=(0,)),
                        slice_sizes=(1,),
                        mode=lax.GatherScatterMode.PROMISE_IN_BOUNDS)
                lg = jnp.where(masks[jj], r + b3vec, lg)
            logit_v[pl.ds(i * 16, 16)] = 1.0 / (1.0 + jnp.exp(-lg))
            return gcarry

        lax.fori_loop(0, C // 16, grp, 0)
        pltpu.sync_copy(logit_v, out_hbm.at[pl.ds(wid * ept + k * C, C)])

    start_g(0, ra0, rb0, sem0)

    def pair(p, pcarry):
        ka = 2 * p
        start_g(ka + 1, ra1, rb1, sem1)
        wait_g(ra0, rb0, sem0)
        do_chunk(ka, ra0, rb0)
        start_g(ka + 2, ra0, rb0, sem0)
        wait_g(ra1, rb1, sem1)
        do_chunk(ka + 1, ra1, rb1)
        return pcarry

    lax.fori_loop(0, (k_chunks - 3) // 2, pair, 0)

    ka = k_chunks - 3
    start_g(ka + 1, ra1, rb1, sem1)
    wait_g(ra0, rb0, sem0)
    do_chunk(ka, ra0, rb0)
    start_g(ka + 2, ra0, rb0, sem0)
    wait_g(ra1, rb1, sem1)
    do_chunk(ka + 1, ra1, rb1)
    wait_g(ra0, rb0, sem0)
    do_chunk(ka + 2, ra0, rb0)


def kernel(x, edge_index, sentence_cls, W_fc0, b_fc0, W1, att_src1, att_dst1,
           b1, W2, att_src2, att_dst2, b2, W_fc2, b_fc2, W_fc3, b_fc3):
    n, d = x.shape
    e = edge_index.shape[1]
    ept = e // NW
    kc = ept // C
    nblk = kc // BCH
    assert e == NW * ept and ept == kc * C and kc == nblk * BCH
    assert n % NS == 0 and n % 16 == 0 and d % 16 == 0 and ept % 16 == 0

    src = edge_index[0].astype(jnp.int32)
    dst = edge_index[1].astype(jnp.int32)
    src4 = src.reshape(NW, nblk, BCH, C)
    dst4 = dst.reshape(NW, nblk, BCH, C)
    src_w = src.reshape(NW, kc, C)
    dst_w = dst.reshape(NW, kc, C)
    f32 = jnp.float32

    mesh = plsc.VectorSubcoreMesh(core_axis_name="c", subcore_axis_name="s")
    sc_params = pltpu.CompilerParams(needs_layout_passes=False)

    gat_scalar = functools.partial(
        pl.kernel,
        out_type=[jax.ShapeDtypeStruct((e,), f32),
                  jax.ShapeDtypeStruct((NW * n,), f32)],
        mesh=mesh,
        compiler_params=sc_params,
        scratch_types=[pltpu.VMEM((n,), f32),
                       pltpu.VMEM((n,), f32),
                       pltpu.VMEM((n,), f32),
                       pltpu.VMEM((ept,), jnp.int32),
                       pltpu.VMEM((ept,), jnp.int32),
                       pltpu.VMEM((ept,), f32)],
    )(_gat_scalar_body)

    gat_agg = functools.partial(
        pl.kernel,
        out_type=jax.ShapeDtypeStruct((NC, n, d), f32),
        mesh=mesh,
        compiler_params=sc_params,
        scratch_types=[pltpu.VMEM((BCH, C), jnp.int32),
                       pltpu.VMEM((BCH, C), jnp.int32),
                       pltpu.VMEM((BCH, C), f32),
                       pltpu.VMEM((C, d), f32),
                       pltpu.VMEM((C, d), f32),
                       pltpu.SemaphoreType.DMA,
                       pltpu.SemaphoreType.DMA,
                       pltpu.VMEM_SHARED((n, d), f32)],
    )(_gat_agg_body)

    edge_mlp = functools.partial(
        pl.kernel,
        out_type=jax.ShapeDtypeStruct((e,), f32),
        mesh=mesh,
        compiler_params=sc_params,
        scratch_types=[pltpu.VMEM((kc, C), jnp.int32),
                       pltpu.VMEM((kc, C), jnp.int32),
                       pltpu.VMEM((C, d), f32),
                       pltpu.VMEM((C, d), f32),
                       pltpu.VMEM((C, d), f32),
                       pltpu.VMEM((C, d), f32),
                       pltpu.SemaphoreType.DMA,
                       pltpu.SemaphoreType.DMA,
                       pltpu.VMEM((d,), f32),
                       pltpu.VMEM((16,), f32),
                       pltpu.VMEM((C,), f32)],
    )(_edge_mlp_body)

    def gat_layer(h, a_s, a_d):
        ex, den = gat_scalar(a_s.reshape(-1), a_d.reshape(-1), src, dst)
        ex4 = ex.reshape(NW, nblk, BCH, C)
        acc = gat_agg(h, ex4, src4, dst4)
        return acc, den.reshape(NW, n)

    h1, as1, ad1 = pl.pallas_call(
        _dense1_body,
        out_shape=[jax.ShapeDtypeStruct((n, d), f32),
                   jax.ShapeDtypeStruct((n, 1), f32),
                   jax.ShapeDtypeStruct((n, 1), f32)],
    )(x, sentence_cls.reshape(1, -1), W_fc0, b_fc0.reshape(1, -1),
      W1[:d], W1[d:], att_src1.reshape(1, -1), att_dst1.reshape(1, -1))

    acc1, den1 = gat_layer(h1, as1, ad1)

    h2, as2, ad2 = pl.pallas_call(
        _mid_body,
        out_shape=[jax.ShapeDtypeStruct((n, d), f32),
                   jax.ShapeDtypeStruct((n, 1), f32),
                   jax.ShapeDtypeStruct((n, 1), f32)],
    )(acc1, den1, h1, as1, ad1, b1.reshape(1, -1), W2,
      att_src2.reshape(1, -1), att_dst2.reshape(1, -1))

    acc2, den2 = gat_layer(h2, as2, ad2)

    a_tab, b_tab = pl.pallas_call(
        _fin_body,
        out_shape=[jax.ShapeDtypeStruct((n, d), f32),
                   jax.ShapeDtypeStruct((n, d), f32)],
    )(acc2, den2, h2, as2, ad2, b2.reshape(1, -1),
      W_fc2[:d], W_fc2[d:], b_fc2.reshape(1, -1))

    probs = edge_mlp(a_tab, b_tab, src_w, dst_w, W_fc3.reshape(-1),
                     jnp.broadcast_to(b_fc3, (16,)))
    return probs.reshape(e, 1)


# speedup vs baseline: 3.7684x; 1.0479x over previous
import functools

import jax
import jax.numpy as jnp
from jax import lax
from jax.experimental import pallas as pl
from jax.experimental.pallas import tpu as pltpu
from jax.experimental.pallas import tpu_sc as plsc

NC = 2
NS = 16
NW = NC * NS
C = 80
BCH = 25


def _dense1_body(x_ref, scls_ref, wfc0_ref, bfc0_ref, w1a_ref, w1b_ref,
                 as_row_ref, ad_row_ref, h_ref, as_ref, ad_ref):
    s = jnp.dot(scls_ref[...], wfc0_ref[...],
                preferred_element_type=jnp.float32) + bfc0_ref[...]
    sh = jnp.dot(s, w1b_ref[...], preferred_element_type=jnp.float32)
    h = jnp.dot(x_ref[...], w1a_ref[...],
                preferred_element_type=jnp.float32) + sh
    h_ref[...] = h
    as_ref[...] = jnp.sum(h * as_row_ref[...], axis=1, keepdims=True)
    ad_ref[...] = jnp.sum(h * ad_row_ref[...], axis=1, keepdims=True)


def _mid_body(accp_ref, denp_ref, h_ref, as_ref, ad_ref, b1_ref, w2_ref,
              as_row_ref, ad_row_ref, h2_ref, as2_ref, ad2_ref):
    acc = accp_ref[0] + accp_ref[1]
    den = jnp.sum(denp_ref[...], axis=0)[:, None]
    a = as_ref[...] + ad_ref[...]
    a = jnp.where(a >= 0.0, a, 0.2 * a)
    exself = jnp.exp(a)
    out = (acc + exself * h_ref[...]) / (den + exself) + b1_ref[...]
    hact = jnp.where(out > 0.0, out, jnp.exp(out) - 1.0)
    h2 = jnp.dot(hact, w2_ref[...], preferred_element_type=jnp.float32)
    h2_ref[...] = h2
    as2_ref[...] = jnp.sum(h2 * as_row_ref[...], axis=1, keepdims=True)
    ad2_ref[...] = jnp.sum(h2 * ad_row_ref[...], axis=1, keepdims=True)


def _fin_body(accp_ref, denp_ref, h_ref, as_ref, ad_ref, b2_ref,
              wfa_ref, wfb_ref, bf2_ref, a_out_ref, b_out_ref):
    acc = accp_ref[0] + accp_ref[1]
    den = jnp.sum(denp_ref[...], axis=0)[:, None]
    a = as_ref[...] + ad_ref[...]
    a = jnp.where(a >= 0.0, a, 0.2 * a)
    exself = jnp.exp(a)
    out = (acc + exself * h_ref[...]) / (den + exself) + b2_ref[...]
    a_out_ref[...] = jnp.dot(out, wfa_ref[...],
                             preferred_element_type=jnp.float32) + bf2_ref[...]
    b_out_ref[...] = jnp.dot(out, wfb_ref[...],
                             preferred_element_type=jnp.float32)


def _gat_scalar_body(asrc_hbm, adst_hbm, src_hbm, dst_hbm,
                     ex_out, den_out,
                     asrc_v, adst_v, den_v, idx_s, idx_d, ex_v):
    cid = lax.axis_index("c")
    sid = lax.axis_index("s")
    wid = cid * NS + sid
    n = asrc_v.shape[0]
    ept = idx_s.shape[0]

    pltpu.sync_copy(asrc_hbm, asrc_v)
    pltpu.sync_copy(adst_hbm, adst_v)
    pltpu.sync_copy(src_hbm.at[pl.ds(wid * ept, ept)], idx_s)
    pltpu.sync_copy(dst_hbm.at[pl.ds(wid * ept, ept)], idx_d)

    zeros16 = jnp.zeros((16,), jnp.float32)

    def zden(i, carry):
        den_v[pl.ds(i * 16, 16)] = zeros16
        return carry

    lax.fori_loop(0, n // 16, zden, 0)

    def grp(i, carry):
        si = idx_s[pl.ds(i * 16, 16)]
        di = idx_d[pl.ds(i * 16, 16)]
        av = plsc.load_gather(asrc_v, [si])
        dv = plsc.load_gather(adst_v, [di])
        al = av + dv
        al = jnp.where(al >= 0.0, al, al * 0.2)
        ex = jnp.exp(al)
        ex_v[pl.ds(i * 16, 16)] = ex
        plsc.addupdate_scatter(den_v, [di], ex)
        return carry

    lax.fori_loop(0, ept // 16, grp, 0)

    pltpu.sync_copy(ex_v, ex_out.at[pl.ds(wid * ept, ept)])
    pltpu.sync_copy(den_v, den_out.at[pl.ds(wid * n, n)])


def _gat_agg_body(h_hbm, ex_hbm, src_hbm, dst_hbm,
                  acc_out,
                  idx_s, idx_d, ex_b, rows0, rows1, rows2,
                  g0, g1, g2, s0, s1, s2, acc_sh):
    cid = lax.axis_index("c")
    sid = lax.axis_index("s")
    wid = cid * NS + sid
    n, d = acc_sh.shape
    nblk = src_hbm.shape[1]
    b8 = (n // NS) // 8 * 8
    tail = n - NS * b8

    zeros16 = jnp.zeros((16,), jnp.float32)

    def zrow(c, carry):
        for j in range(d // 16):
            rows0[c, pl.ds(j * 16, 16)] = zeros16
        return carry

    lax.fori_loop(0, C, zrow, 0)

    full = b8 // C
    rem = b8 - full * C
    for z in range(full):
        pltpu.sync_copy(rows0, acc_sh.at[pl.ds(sid * b8 + z * C, C)])
    if rem:
        pltpu.sync_copy(rows0.at[pl.ds(0, rem)],
                        acc_sh.at[pl.ds(sid * b8 + full * C, rem)])
    if tail:
        @pl.when(sid == 0)
        def _zero_tail():
            pltpu.sync_copy(rows0.at[pl.ds(0, tail)],
                            acc_sh.at[pl.ds(NS * b8, tail)])
    plsc.subcore_barrier()

    rows = [rows0, rows1, rows2]
    gsem = [g0, g1, g2]
    ssem = [s0, s1, s2]

    def start_g(k, i):
        pltpu.async_copy(h_hbm.at[idx_s.at[k]], rows[i], gsem[i])

    def wait_g(i):
        pltpu.make_async_copy(h_hbm.at[idx_s.at[0]], rows[i], gsem[i]).wait()

    def start_s(k, i):
        pltpu.async_copy(rows[i], acc_sh.at[idx_d.at[k]], ssem[i], add=True)

    def wait_s(i):
        pltpu.make_async_copy(rows[i], acc_sh.at[idx_d.at[0]],
                              ssem[i]).wait()

    def scale(k, i):
        def sgrp(g, scarry):
            ex16 = ex_b[k, pl.ds(g * 16, 16)]
            for jj in range(16):
                e = ex16[jj]
                c = g * 16 + jj
                for j in range(d // 16):
                    rows[i][c, pl.ds(j * 16, 16)] = (
                        rows[i][c, pl.ds(j * 16, 16)] * e)
            return scarry

        lax.fori_loop(0, C // 16, sgrp, 0)

    def blk(b, carry):
        pltpu.sync_copy(src_hbm.at[wid, b], idx_s)
        pltpu.sync_copy(dst_hbm.at[wid, b], idx_d)
        pltpu.sync_copy(ex_hbm.at[wid, b], ex_b)

        start_g(0, 0)
        start_g(1, 1)
        wait_g(0)
        scale(0, 0)
        start_s(0, 0)
        start_g(2, 2)

        def triple(t, tcarry):
            for dk in range(3):
                k = 1 + 3 * t + dk
                cur = (1 + dk) % 3
                old = dk
                wait_g(cur)
                scale(k, cur)
                start_s(k, cur)
                wait_s(old)
                start_g(k + 2, old)
            return tcarry

        lax.fori_loop(0, (BCH - 4) // 3, triple, 0)

        ka = BCH - 3
        wait_g(ka % 3)
        scale(ka, ka % 3)
        start_s(ka, ka % 3)
        wait_s((ka + 2) % 3)
        start_g(ka + 2, (ka + 2) % 3)
        wait_g((ka + 1) % 3)
        scale(ka + 1, (ka + 1) % 3)
        start_s(ka + 1, (ka + 1) % 3)
        wait_s((ka + 3) % 3)
        wait_g((ka + 2) % 3)
        scale(ka + 2, (ka + 2) % 3)
        start_s(ka + 2, (ka + 2) % 3)
        wait_s((ka + 4) % 3)
        wait_s((ka + 2) % 3)
        return carry

    lax.fori_loop(0, nblk, blk, 0)

    plsc.subcore_barrier()
    pltpu.sync_copy(acc_sh.at[pl.ds(sid * b8, b8)],
                    acc_out.at[cid, pl.ds(sid * b8, b8)])
    if tail:
        @pl.when(sid == 0)
        def _write_tail():
            pltpu.sync_copy(acc_sh.at[pl.ds(NS * b8, tail)],
                            acc_out.at[cid, pl.ds(NS * b8, tail)])


def _edge_mlp_body(a_hbm, b_hbm, src_hbm, dst_hbm, w3_hbm, b3_hbm,
                   out_hbm,
                   idx_s, idx_d, ra0, rb0, ra1, rb1, sem0, sem1,
                   w3_v, b3_v, logit_v):
    cid = lax.axis_index("c")
    sid = lax.axis_index("s")
    wid = cid * NS + sid
    d = ra0.shape[1]
    k_chunks = idx_s.shape[0]
    ept = k_chunks * C

    pltpu.sync_copy(src_hbm.at[wid], idx_s)
    pltpu.sync_copy(dst_hbm.at[wid], idx_d)
    pltpu.sync_copy(w3_hbm, w3_v)
    pltpu.sync_copy(b3_hbm, b3_v)

    w3v = [w3_v[pl.ds(j * 16, 16)] for j in range(d // 16)]
    b3vec = b3_v[...]
    iota16 = lax.iota(jnp.int32, 16)
    perms = [iota16 ^ 8, iota16 ^ 4, iota16 ^ 2, iota16 ^ 1]
    masks = [iota16 == jj for jj in range(16)]

    def start_g(k, ra, rb, sem):
        pltpu.async_copy(a_hbm.at[idx_s.at[k]], ra, sem)
        pltpu.async_copy(b_hbm.at[idx_d.at[k]], rb, sem)

    def wait_g(ra, rb, sem):
        pltpu.make_async_copy(a_hbm.at[idx_s.at[0]], ra, sem).wait()
        pltpu.make_async_copy(b_hbm.at[idx_d.at[0]], rb, sem).wait()

    def do_chunk(k, ra, rb):
        def grp(i, gcarry):
            lg = b3vec
            for jj in range(16):
                c = i * 16 + jj
                acc0 = jnp.zeros((16,), jnp.float32)
                acc1 = jnp.zeros((16,), jnp.float32)
                for j in range(d // 16):
                    va = ra[c, pl.ds(j * 16, 16)]
                    vb = rb[c, pl.ds(j * 16, 16)]
                    v = jnp.maximum(va + vb, 0.0) * w3v[j]
                    if j % 2 == 0:
                        acc0 = acc0 + v
                    else:
                        acc1 = acc1 + v
                r = acc0 + acc1
                for p in perms:
                    r = r + lax.gather(
                        r, p[:, None],
                        lax.GatherDimensionNumbers(
                            offset_dims=(), collapsed_slice_dims=(0,),
                            start_index_map=(0,)),
                        slice_sizes=(1,),
                        mode=lax.GatherScatterMode.PROMISE_IN_BOUNDS)
                lg = jnp.where(masks[jj], r + b3vec, lg)
            logit_v[pl.ds(i * 16, 16)] = 1.0 / (1.0 + jnp.exp(-lg))
            return gcarry

        lax.fori_loop(0, C // 16, grp, 0)
        pltpu.sync_copy(logit_v, out_hbm.at[pl.ds(wid * ept + k * C, C)])

    start_g(0, ra0, rb0, sem0)

    def pair(p, pcarry):
        ka = 2 * p
        start_g(ka + 1, ra1, rb1, sem1)
        wait_g(ra0, rb0, sem0)
        do_chunk(ka, ra0, rb0)
        start_g(ka + 2, ra0, rb0, sem0)
        wait_g(ra1, rb1, sem1)
        do_chunk(ka + 1, ra1, rb1)
        return pcarry

    lax.fori_loop(0, (k_chunks - 3) // 2, pair, 0)

    ka = k_chunks - 3
    start_g(ka + 1, ra1, rb1, sem1)
    wait_g(ra0, rb0, sem0)
    do_chunk(ka, ra0, rb0)
    start_g(ka + 2, ra0, rb0, sem0)
    wait_g(ra1, rb1, sem1)
    do_chunk(ka + 1, ra1, rb1)
    wait_g(ra0, rb0, sem0)
    do_chunk(ka + 2, ra0, rb0)


def kernel(x, edge_index, sentence_cls, W_fc0, b_fc0, W1, att_src1, att_dst1,
           b1, W2, att_src2, att_dst2, b2, W_fc2, b_fc2, W_fc3, b_fc3):
    n, d = x.shape
    e = edge_index.shape[1]
    ept = e // NW
    kc = ept // C
    nblk = kc // BCH
    assert e == NW * ept and ept == kc * C and kc == nblk * BCH
    assert n % NS == 0 and n % 16 == 0 and d % 16 == 0 and ept % 16 == 0

    src = edge_index[0].astype(jnp.int32)
    dst = edge_index[1].astype(jnp.int32)
    src4 = src.reshape(NW, nblk, BCH, C)
    dst4 = dst.reshape(NW, nblk, BCH, C)
    src_w = src.reshape(NW, kc, C)
    dst_w = dst.reshape(NW, kc, C)
    f32 = jnp.float32

    mesh = plsc.VectorSubcoreMesh(core_axis_name="c", subcore_axis_name="s")
    sc_params = pltpu.CompilerParams(needs_layout_passes=False)

    gat_scalar = functools.partial(
        pl.kernel,
        out_type=[jax.ShapeDtypeStruct((e,), f32),
                  jax.ShapeDtypeStruct((NW * n,), f32)],
        mesh=mesh,
        compiler_params=sc_params,
        scratch_types=[pltpu.VMEM((n,), f32),
                       pltpu.VMEM((n,), f32),
                       pltpu.VMEM((n,), f32),
                       pltpu.VMEM((ept,), jnp.int32),
                       pltpu.VMEM((ept,), jnp.int32),
                       pltpu.VMEM((ept,), f32)],
    )(_gat_scalar_body)

    gat_agg = functools.partial(
        pl.kernel,
        out_type=jax.ShapeDtypeStruct((NC, n, d), f32),
        mesh=mesh,
        compiler_params=sc_params,
        scratch_types=[pltpu.VMEM((BCH, C), jnp.int32),
                       pltpu.VMEM((BCH, C), jnp.int32),
                       pltpu.VMEM((BCH, C), f32),
                       pltpu.VMEM((C, d), f32),
                       pltpu.VMEM((C, d), f32),
                       pltpu.VMEM((C, d), f32),
                       pltpu.SemaphoreType.DMA,
                       pltpu.SemaphoreType.DMA,
                       pltpu.SemaphoreType.DMA,
                       pltpu.SemaphoreType.DMA,
                       pltpu.SemaphoreType.DMA,
                       pltpu.SemaphoreType.DMA,
                       pltpu.VMEM_SHARED((n, d), f32)],
    )(_gat_agg_body)

    edge_mlp = functools.partial(
        pl.kernel,
        out_type=jax.ShapeDtypeStruct((e,), f32),
        mesh=mesh,
        compiler_params=sc_params,
        scratch_types=[pltpu.VMEM((kc, C), jnp.int32),
                       pltpu.VMEM((kc, C), jnp.int32),
                       pltpu.VMEM((C, d), f32),
                       pltpu.VMEM((C, d), f32),
                       pltpu.VMEM((C, d), f32),
                       pltpu.VMEM((C, d), f32),
                       pltpu.SemaphoreType.DMA,
                       pltpu.SemaphoreType.DMA,
                       pltpu.VMEM((d,), f32),
                       pltpu.VMEM((16,), f32),
                       pltpu.VMEM((C,), f32)],
    )(_edge_mlp_body)

    def gat_layer(h, a_s, a_d):
        ex, den = gat_scalar(a_s.reshape(-1), a_d.reshape(-1), src, dst)
        ex4 = ex.reshape(NW, nblk, BCH, C)
        acc = gat_agg(h, ex4, src4, dst4)
        return acc, den.reshape(NW, n)

    h1, as1, ad1 = pl.pallas_call(
        _dense1_body,
        out_shape=[jax.ShapeDtypeStruct((n, d), f32),
                   jax.ShapeDtypeStruct((n, 1), f32),
                   jax.ShapeDtypeStruct((n, 1), f32)],
    )(x, sentence_cls.reshape(1, -1), W_fc0, b_fc0.reshape(1, -1),
      W1[:d], W1[d:], att_src1.reshape(1, -1), att_dst1.reshape(1, -1))

    acc1, den1 = gat_layer(h1, as1, ad1)

    h2, as2, ad2 = pl.pallas_call(
        _mid_body,
        out_shape=[jax.ShapeDtypeStruct((n, d), f32),
                   jax.ShapeDtypeStruct((n, 1), f32),
                   jax.ShapeDtypeStruct((n, 1), f32)],
    )(acc1, den1, h1, as1, ad1, b1.reshape(1, -1), W2,
      att_src2.reshape(1, -1), att_dst2.reshape(1, -1))

    acc2, den2 = gat_layer(h2, as2, ad2)

    a_tab, b_tab = pl.pallas_call(
        _fin_body,
        out_shape=[jax.ShapeDtypeStruct((n, d), f32),
                   jax.ShapeDtypeStruct((n, d), f32)],
    )(acc2, den2, h2, as2, ad2, b2.reshape(1, -1),
      W_fc2[:d], W_fc2[d:], b_fc2.reshape(1, -1))

    probs = edge_mlp(a_tab, b_tab, src_w, dst_w, W_fc3.reshape(-1),
                     jnp.broadcast_to(b_fc3, (16,)))
    return probs.reshape(e, 1)


# speedup vs baseline: 3.8413x; 1.0194x over previous
import functools

import jax
import jax.numpy as jnp
from jax import lax
from jax.experimental import pallas as pl
from jax.experimental.pallas import tpu as pltpu
from jax.experimental.pallas import tpu_sc as plsc

NC = 2
NS = 16
NW = NC * NS
C = 80
BCH = 25


def _dense1_body(x_ref, scls_ref, wfc0_ref, bfc0_ref, w1a_ref, w1b_ref,
                 as_row_ref, ad_row_ref, h_ref, as_ref, ad_ref):
    s = jnp.dot(scls_ref[...], wfc0_ref[...],
                preferred_element_type=jnp.float32) + bfc0_ref[...]
    sh = jnp.dot(s, w1b_ref[...], preferred_element_type=jnp.float32)
    h = jnp.dot(x_ref[...], w1a_ref[...],
                preferred_element_type=jnp.float32) + sh
    h_ref[...] = h
    as_ref[...] = jnp.sum(h * as_row_ref[...], axis=1, keepdims=True)
    ad_ref[...] = jnp.sum(h * ad_row_ref[...], axis=1, keepdims=True)


def _mid_body(accp_ref, denp_ref, h_ref, as_ref, ad_ref, b1_ref, w2_ref,
              as_row_ref, ad_row_ref, h2_ref, as2_ref, ad2_ref):
    acc = accp_ref[0] + accp_ref[1]
    den = jnp.sum(denp_ref[...], axis=0)[:, None]
    a = as_ref[...] + ad_ref[...]
    a = jnp.where(a >= 0.0, a, 0.2 * a)
    exself = jnp.exp(a)
    out = (acc + exself * h_ref[...]) / (den + exself) + b1_ref[...]
    hact = jnp.where(out > 0.0, out, jnp.exp(out) - 1.0)
    h2 = jnp.dot(hact, w2_ref[...], preferred_element_type=jnp.float32)
    h2_ref[...] = h2
    as2_ref[...] = jnp.sum(h2 * as_row_ref[...], axis=1, keepdims=True)
    ad2_ref[...] = jnp.sum(h2 * ad_row_ref[...], axis=1, keepdims=True)


def _fin_body(accp_ref, denp_ref, h_ref, as_ref, ad_ref, b2_ref,
              wfa_ref, wfb_ref, bf2_ref, a_out_ref, b_out_ref):
    acc = accp_ref[0] + accp_ref[1]
    den = jnp.sum(denp_ref[...], axis=0)[:, None]
    a = as_ref[...] + ad_ref[...]
    a = jnp.where(a >= 0.0, a, 0.2 * a)
    exself = jnp.exp(a)
    out = (acc + exself * h_ref[...]) / (den + exself) + b2_ref[...]
    a_out_ref[...] = jnp.dot(out, wfa_ref[...],
                             preferred_element_type=jnp.float32) + bf2_ref[...]
    b_out_ref[...] = jnp.dot(out, wfb_ref[...],
                             preferred_element_type=jnp.float32)


def _gat_scalar_body(asrc_hbm, adst_hbm, src_hbm, dst_hbm,
                     ex_out, den_out,
                     asrc_v, adst_v, den_v, idx_s, idx_d, ex_v):
    cid = lax.axis_index("c")
    sid = lax.axis_index("s")
    wid = cid * NS + sid
    n = asrc_v.shape[0]
    ept = idx_s.shape[0]

    pltpu.sync_copy(asrc_hbm, asrc_v)
    pltpu.sync_copy(adst_hbm, adst_v)
    pltpu.sync_copy(src_hbm.at[pl.ds(wid * ept, ept)], idx_s)
    pltpu.sync_copy(dst_hbm.at[pl.ds(wid * ept, ept)], idx_d)

    zeros16 = jnp.zeros((16,), jnp.float32)

    def zden(i, carry):
        den_v[pl.ds(i * 16, 16)] = zeros16
        return carry

    lax.fori_loop(0, n // 16, zden, 0)

    def grp(i, carry):
        si = idx_s[pl.ds(i * 16, 16)]
        di = idx_d[pl.ds(i * 16, 16)]
        av = plsc.load_gather(asrc_v, [si])
        dv = plsc.load_gather(adst_v, [di])
        al = av + dv
        al = jnp.where(al >= 0.0, al, al * 0.2)
        ex = jnp.exp(al)
        ex_v[pl.ds(i * 16, 16)] = ex
        plsc.addupdate_scatter(den_v, [di], ex)
        return carry

    lax.fori_loop(0, ept // 16, grp, 0)

    pltpu.sync_copy(ex_v, ex_out.at[pl.ds(wid * ept, ept)])
    pltpu.sync_copy(den_v, den_out.at[pl.ds(wid * n, n)])


def _gat_agg_body(h_hbm, ex_hbm, src_hbm, dst_hbm,
                  acc_out,
                  idx_s, idx_d, ex_b, rows0, rows1, rows2,
                  g0, g1, g2, s0, s1, s2, acc_sh):
    cid = lax.axis_index("c")
    sid = lax.axis_index("s")
    wid = cid * NS + sid
    n, d = acc_sh.shape
    nblk = src_hbm.shape[1]
    b8 = (n // NS) // 8 * 8
    tail = n - NS * b8

    zeros16 = jnp.zeros((16,), jnp.float32)

    def zrow(c, carry):
        for j in range(d // 16):
            rows0[c, pl.ds(j * 16, 16)] = zeros16
        return carry

    lax.fori_loop(0, C, zrow, 0)

    full = b8 // C
    rem = b8 - full * C
    for z in range(full):
        pltpu.sync_copy(rows0, acc_sh.at[pl.ds(sid * b8 + z * C, C)])
    if rem:
        pltpu.sync_copy(rows0.at[pl.ds(0, rem)],
                        acc_sh.at[pl.ds(sid * b8 + full * C, rem)])
    if tail:
        @pl.when(sid == 0)
        def _zero_tail():
            pltpu.sync_copy(rows0.at[pl.ds(0, tail)],
                            acc_sh.at[pl.ds(NS * b8, tail)])
    plsc.subcore_barrier()

    rows = [rows0, rows1, rows2]
    gsem = [g0, g1, g2]
    ssem = [s0, s1, s2]

    def start_g(k, i):
        pltpu.async_copy(h_hbm.at[idx_s.at[k]], rows[i], gsem[i])

    def wait_g(i):
        pltpu.make_async_copy(h_hbm.at[idx_s.at[0]], rows[i], gsem[i]).wait()

    def start_s(k, i):
        pltpu.async_copy(rows[i], acc_sh.at[idx_d.at[k]], ssem[i], add=True)

    def wait_s(i):
        pltpu.make_async_copy(rows[i], acc_sh.at[idx_d.at[0]],
                              ssem[i]).wait()

    def scale(k, i):
        def sgrp(g, scarry):
            ex16 = ex_b[k, pl.ds(g * 16, 16)]
            for jj in range(16):
                e = ex16[jj]
                c = g * 16 + jj
                for j in range(d // 16):
                    rows[i][c, pl.ds(j * 16, 16)] = (
                        rows[i][c, pl.ds(j * 16, 16)] * e)
            return scarry

        lax.fori_loop(0, C // 16, sgrp, 0)

    def blk(b, carry):
        pltpu.sync_copy(src_hbm.at[wid, b], idx_s)
        pltpu.sync_copy(dst_hbm.at[wid, b], idx_d)
        pltpu.sync_copy(ex_hbm.at[wid, b], ex_b)

        start_g(0, 0)
        start_g(1, 1)
        wait_g(0)
        scale(0, 0)
        start_s(0, 0)
        start_g(2, 2)

        def triple(t, tcarry):
            for dk in range(3):
                k = 1 + 3 * t + dk
                cur = (1 + dk) % 3
                old = dk
                wait_g(cur)
                scale(k, cur)
                start_s(k, cur)
                wait_s(old)
                start_g(k + 2, old)
            return tcarry

        lax.fori_loop(0, (BCH - 4) // 3, triple, 0)

        ka = BCH - 3
        wait_g(ka % 3)
        scale(ka, ka % 3)
        start_s(ka, ka % 3)
        wait_s((ka + 2) % 3)
        start_g(ka + 2, (ka + 2) % 3)
        wait_g((ka + 1) % 3)
        scale(ka + 1, (ka + 1) % 3)
        start_s(ka + 1, (ka + 1) % 3)
        wait_s((ka + 3) % 3)
        wait_g((ka + 2) % 3)
        scale(ka + 2, (ka + 2) % 3)
        start_s(ka + 2, (ka + 2) % 3)
        wait_s((ka + 4) % 3)
        wait_s((ka + 2) % 3)
        return carry

    lax.fori_loop(0, nblk, blk, 0)

    plsc.subcore_barrier()
    pltpu.sync_copy(acc_sh.at[pl.ds(sid * b8, b8)],
                    acc_out.at[cid, pl.ds(sid * b8, b8)])
    if tail:
        @pl.when(sid == 0)
        def _write_tail():
            pltpu.sync_copy(acc_sh.at[pl.ds(NS * b8, tail)],
                            acc_out.at[cid, pl.ds(NS * b8, tail)])


def _edge_mlp_body(a_hbm, b_hbm, src_hbm, dst_hbm, w3_hbm, b3_hbm,
                   out_hbm,
                   idx_s, idx_d, ra0, rb0, ra1, rb1, ra2, rb2,
                   sem0, sem1, sem2, w3_v, b3_v, logit_v):
    cid = lax.axis_index("c")
    sid = lax.axis_index("s")
    wid = cid * NS + sid
    d = ra0.shape[1]
    k_chunks = idx_s.shape[0]
    ept = k_chunks * C
    ras = [ra0, ra1, ra2]
    rbs = [rb0, rb1, rb2]
    sems = [sem0, sem1, sem2]

    pltpu.sync_copy(src_hbm.at[wid], idx_s)
    pltpu.sync_copy(dst_hbm.at[wid], idx_d)
    pltpu.sync_copy(w3_hbm, w3_v)
    pltpu.sync_copy(b3_hbm, b3_v)

    w3v = [w3_v[pl.ds(j * 16, 16)] for j in range(d // 16)]
    b3vec = b3_v[...]
    iota16 = lax.iota(jnp.int32, 16)
    perms = [iota16 ^ 8, iota16 ^ 4, iota16 ^ 2, iota16 ^ 1]
    masks = [iota16 == jj for jj in range(16)]

    def start_g(k, i):
        pltpu.async_copy(a_hbm.at[idx_s.at[k]], ras[i], sems[i])
        pltpu.async_copy(b_hbm.at[idx_d.at[k]], rbs[i], sems[i])

    def wait_g(i):
        pltpu.make_async_copy(a_hbm.at[idx_s.at[0]], ras[i], sems[i]).wait()
        pltpu.make_async_copy(b_hbm.at[idx_d.at[0]], rbs[i], sems[i]).wait()

    def do_chunk(k, i):
        ra = ras[i]
        rb = rbs[i]
        def grp(i, gcarry):
            lg = b3vec
            for jj in range(16):
                c = i * 16 + jj
                acc0 = jnp.zeros((16,), jnp.float32)
                acc1 = jnp.zeros((16,), jnp.float32)
                for j in range(d // 16):
                    va = ra[c, pl.ds(j * 16, 16)]
                    vb = rb[c, pl.ds(j * 16, 16)]
                    v = jnp.maximum(va + vb, 0.0) * w3v[j]
                    if j % 2 == 0:
                        acc0 = acc0 + v
                    else:
                        acc1 = acc1 + v
                r = acc0 + acc1
                for p in perms:
                    r = r + lax.gather(
                        r, p[:, None],
                        lax.GatherDimensionNumbers(
                            offset_dims=(), collapsed_slice_dims=(0,),
                            start_index_map=(0,)),
                        slice_sizes=(1,),
                        mode=lax.GatherScatterMode.PROMISE_IN_BOUNDS)
                lg = jnp.where(masks[jj], r + b3vec, lg)
            logit_v[pl.ds(i * 16, 16)] = 1.0 / (1.0 + jnp.exp(-lg))
            return gcarry

        lax.fori_loop(0, C // 16, grp, 0)
        pltpu.sync_copy(logit_v, out_hbm.at[pl.ds(wid * ept + k * C, C)])

    start_g(0, 0)
    start_g(1, 1)
    wait_g(0)
    start_g(2, 2)
    do_chunk(0, 0)

    ntrip = (k_chunks - 5) // 3

    def triple(t, tcarry):
        for dk in range(3):
            k = 1 + 3 * t + dk
            cur = (1 + dk) % 3
            wait_g(cur)
            start_g(k + 2, dk)
            do_chunk(k, cur)
        return tcarry

    lax.fori_loop(0, ntrip, triple, 0)

    for k in range(3 * ntrip + 1, k_chunks):
        cur = k % 3
        wait_g(cur)
        if k + 2 < k_chunks:
            start_g(k + 2, (k + 2) % 3)
        do_chunk(k, cur)


def kernel(x, edge_index, sentence_cls, W_fc0, b_fc0, W1, att_src1, att_dst1,
           b1, W2, att_src2, att_dst2, b2, W_fc2, b_fc2, W_fc3, b_fc3):
    n, d = x.shape
    e = edge_index.shape[1]
    ept = e // NW
    kc = ept // C
    nblk = kc // BCH
    assert e == NW * ept and ept == kc * C and kc == nblk * BCH
    assert n % NS == 0 and n % 16 == 0 and d % 16 == 0 and ept % 16 == 0

    src = edge_index[0].astype(jnp.int32)
    dst = edge_index[1].astype(jnp.int32)
    src4 = src.reshape(NW, nblk, BCH, C)
    dst4 = dst.reshape(NW, nblk, BCH, C)
    src_w = src.reshape(NW, kc, C)
    dst_w = dst.reshape(NW, kc, C)
    f32 = jnp.float32

    mesh = plsc.VectorSubcoreMesh(core_axis_name="c", subcore_axis_name="s")
    sc_params = pltpu.CompilerParams(needs_layout_passes=False)

    gat_scalar = functools.partial(
        pl.kernel,
        out_type=[jax.ShapeDtypeStruct((e,), f32),
                  jax.ShapeDtypeStruct((NW * n,), f32)],
        mesh=mesh,
        compiler_params=sc_params,
        scratch_types=[pltpu.VMEM((n,), f32),
                       pltpu.VMEM((n,), f32),
                       pltpu.VMEM((n,), f32),
                       pltpu.VMEM((ept,), jnp.int32),
                       pltpu.VMEM((ept,), jnp.int32),
                       pltpu.VMEM((ept,), f32)],
    )(_gat_scalar_body)

    gat_agg = functools.partial(
        pl.kernel,
        out_type=jax.ShapeDtypeStruct((NC, n, d), f32),
        mesh=mesh,
        compiler_params=sc_params,
        scratch_types=[pltpu.VMEM((BCH, C), jnp.int32),
                       pltpu.VMEM((BCH, C), jnp.int32),
                       pltpu.VMEM((BCH, C), f32),
                       pltpu.VMEM((C, d), f32),
                       pltpu.VMEM((C, d), f32),
                       pltpu.VMEM((C, d), f32),
                       pltpu.SemaphoreType.DMA,
                       pltpu.SemaphoreType.DMA,
                       pltpu.SemaphoreType.DMA,
                       pltpu.SemaphoreType.DMA,
                       pltpu.SemaphoreType.DMA,
                       pltpu.SemaphoreType.DMA,
                       pltpu.VMEM_SHARED((n, d), f32)],
    )(_gat_agg_body)

    edge_mlp = functools.partial(
        pl.kernel,
        out_type=jax.ShapeDtypeStruct((e,), f32),
        mesh=mesh,
        compiler_params=sc_params,
        scratch_types=[pltpu.VMEM((kc, C), jnp.int32),
                       pltpu.VMEM((kc, C), jnp.int32),
                       pltpu.VMEM((C, d), f32),
                       pltpu.VMEM((C, d), f32),
                       pltpu.VMEM((C, d), f32),
                       pltpu.VMEM((C, d), f32),
                       pltpu.VMEM((C, d), f32),
                       pltpu.VMEM((C, d), f32),
                       pltpu.SemaphoreType.DMA,
                       pltpu.SemaphoreType.DMA,
                       pltpu.SemaphoreType.DMA,
                       pltpu.VMEM((d,), f32),
                       pltpu.VMEM((16,), f32),
                       pltpu.VMEM((C,), f32)],
    )(_edge_mlp_body)

    def gat_layer(h, a_s, a_d):
        ex, den = gat_scalar(a_s.reshape(-1), a_d.reshape(-1), src, dst)
        ex4 = ex.reshape(NW, nblk, BCH, C)
        acc = gat_agg(h, ex4, src4, dst4)
        return acc, den.reshape(NW, n)

    h1, as1, ad1 = pl.pallas_call(
        _dense1_body,
        out_shape=[jax.ShapeDtypeStruct((n, d), f32),
                   jax.ShapeDtypeStruct((n, 1), f32),
                   jax.ShapeDtypeStruct((n, 1), f32)],
    )(x, sentence_cls.reshape(1, -1), W_fc0, b_fc0.reshape(1, -1),
      W1[:d], W1[d:], att_src1.reshape(1, -1), att_dst1.reshape(1, -1))

    acc1, den1 = gat_layer(h1, as1, ad1)

    h2, as2, ad2 = pl.pallas_call(
        _mid_body,
        out_shape=[jax.ShapeDtypeStruct((n, d), f32),
                   jax.ShapeDtypeStruct((n, 1), f32),
                   jax.ShapeDtypeStruct((n, 1), f32)],
    )(acc1, den1, h1, as1, ad1, b1.reshape(1, -1), W2,
      att_src2.reshape(1, -1), att_dst2.reshape(1, -1))

    acc2, den2 = gat_layer(h2, as2, ad2)

    a_tab, b_tab = pl.pallas_call(
        _fin_body,
        out_shape=[jax.ShapeDtypeStruct((n, d), f32),
                   jax.ShapeDtypeStruct((n, d), f32)],
    )(acc2, den2, h2, as2, ad2, b2.reshape(1, -1),
      W_fc2[:d], W_fc2[d:], b_fc2.reshape(1, -1))

    probs = edge_mlp(a_tab, b_tab, src_w, dst_w, W_fc3.reshape(-1),
                     jnp.broadcast_to(b_fc3, (16,)))
    return probs.reshape(e, 1)
